# Initial kernel scaffold; baseline (speedup 1.0000x reference)
import functools

import jax
import jax.numpy as jnp
from jax import lax
from jax.experimental import pallas as pl
from jax.experimental.pallas import tpu as pltpu
from jax.experimental.pallas import tpu_sc as plsc

_N = 10000
_E = 320000
_NC = 2
_NS = 16
_K = 80
_R = 1000
_RPT = _N // _NS


def _make_deg():
    ept = _E // (_NC * _NS)
    nch = ept // _K

    @functools.partial(
        pl.kernel,
        out_type=jax.ShapeDtypeStruct((_NC, _N, 16), jnp.float32),
        mesh=plsc.VectorSubcoreMesh(core_axis_name="c", subcore_axis_name="s"),
        scratch_types=[
            pltpu.VMEM((_K, 16), jnp.float32),
            pltpu.VMEM((nch, _K), jnp.int32),
            pltpu.VMEM_SHARED((_N, 16), jnp.float32),
        ],
    )
    def deg(dst_hbm, ones_hbm, zeros_hbm, out_hbm, ones_v, idx_v, table):
        c = lax.axis_index("c")
        s = lax.axis_index("s")
        pltpu.sync_copy(ones_hbm, ones_v)
        pltpu.sync_copy(zeros_hbm.at[pl.ds(s * _RPT, _RPT), :16],
                        table.at[pl.ds(s * _RPT, _RPT)])
        pltpu.sync_copy(dst_hbm.at[c * _NS + s], idx_v)
        plsc.subcore_barrier()

        def chunk(i, carry):
            pltpu.sync_copy(ones_v, table.at[idx_v.at[i]], add=True)
            return carry

        lax.fori_loop(0, nch, chunk, 0)
        plsc.subcore_barrier()
        pltpu.sync_copy(table.at[pl.ds(s * _RPT, _RPT)],
                        out_hbm.at[c, pl.ds(s * _RPT, _RPT)])

    return deg


def _make_spmm(table_rows, epc):
    ept = epc // _NS
    nch = ept // _K

    @functools.partial(
        pl.kernel,
        out_type=jax.ShapeDtypeStruct((_NC, _N, 128), jnp.float32),
        mesh=plsc.VectorSubcoreMesh(core_axis_name="c", subcore_axis_name="s"),
        scratch_types=[
            pltpu.VMEM((nch, _K), jnp.int32),
            pltpu.VMEM((nch, _K), jnp.int32),
            pltpu.VMEM((_K, 128), jnp.float32),
            pltpu.VMEM_SHARED((_N, 128), jnp.float32),
            pltpu.SemaphoreType.DMA,
        ],
    )
    def spmm(tab_hbm, src_hbm, dst_hbm, zeros_hbm, out_hbm,
             idx_s, idx_d, rows, acc, sem):
        c = lax.axis_index("c")
        s = lax.axis_index("s")
        pltpu.sync_copy(zeros_hbm.at[pl.ds(s * _RPT, _RPT)],
                        acc.at[pl.ds(s * _RPT, _RPT)])
        pltpu.sync_copy(src_hbm.at[c, s], idx_s)
        pltpu.sync_copy(dst_hbm.at[c, s], idx_d)
        plsc.subcore_barrier()

        def chunk(i, carry):
            pltpu.async_copy(tab_hbm.at[idx_s.at[i]], rows, sem).wait()
            pltpu.sync_copy(rows, acc.at[idx_d.at[i]], add=True)
            return carry

        lax.fori_loop(0, nch, chunk, 0)
        plsc.subcore_barrier()
        pltpu.sync_copy(acc.at[pl.ds(s * _RPT, _RPT)],
                        out_hbm.at[c, pl.ds(s * _RPT, _RPT)])

    return spmm


_deg_k = _make_deg()
_spmm_l1 = _make_spmm(_NC * _N, _E)
_spmm_l2 = _make_spmm(_N, _E // _NC)


def _dinv_of(degp_blk):
    deg = degp_blk[0, :, 0] + degp_blk[1, :, 0] + 1.0
    return lax.rsqrt(deg)[:, None]


def _tc1(x, W1, degp):
    def body(x_ref, w_ref, degp_ref, out_ref):
        dinv = _dinv_of(degp_ref[...])
        h = jnp.dot(x_ref[...], w_ref[...], preferred_element_type=jnp.float32)
        hs = h * dinv
        out_ref[0] = hs[:, :128]
        out_ref[1] = hs[:, 128:]

    return pl.pallas_call(
        body,
        grid=(_N // _R,),
        in_specs=[
            pl.BlockSpec((_R, 128), lambda g: (g, 0)),
            pl.BlockSpec((128, 256), lambda g: (0, 0)),
            pl.BlockSpec((_NC, _R, 16), lambda g: (0, g, 0)),
        ],
        out_specs=pl.BlockSpec((_NC, _R, 128), lambda g: (0, g, 0)),
        out_shape=jax.ShapeDtypeStruct((_NC, _N, 128), jnp.float32),
    )(x, W1, degp)


def _tc2(agg1, hs1, degp, b1r, W2r):
    def body(agg_ref, hs_ref, degp_ref, b_ref, w_ref, out_ref):
        dinv = _dinv_of(degp_ref[...])
        y0 = jnp.maximum(dinv * (agg_ref[0] + hs_ref[0]) + b_ref[0], 0.0)
        y1 = jnp.maximum(dinv * (agg_ref[1] + hs_ref[1]) + b_ref[1], 0.0)
        h2 = (jnp.dot(y0, w_ref[0], preferred_element_type=jnp.float32)
              + jnp.dot(y1, w_ref[1], preferred_element_type=jnp.float32))
        out_ref[...] = h2 * dinv

    return pl.pallas_call(
        body,
        grid=(_N // _R,),
        in_specs=[
            pl.BlockSpec((_NC, _R, 128), lambda g: (0, g, 0)),
            pl.BlockSpec((_NC, _R, 128), lambda g: (0, g, 0)),
            pl.BlockSpec((_NC, _R, 16), lambda g: (0, g, 0)),
            pl.BlockSpec((2, 128), lambda g: (0, 0)),
            pl.BlockSpec((2, 128, 128), lambda g: (0, 0, 0)),
        ],
        out_specs=pl.BlockSpec((_R, 128), lambda g: (g, 0)),
        out_shape=jax.ShapeDtypeStruct((_N, 128), jnp.float32),
    )(agg1, hs1, degp, b1r, W2r)


def _tc3(agg2, hs2, degp, b2r):
    def body(agg_ref, hs_ref, degp_ref, b_ref, out_ref):
        dinv = _dinv_of(degp_ref[...])
        msg = agg_ref[0] + agg_ref[1] + hs_ref[...]
        out_ref[...] = jnp.maximum(dinv * msg + b_ref[0], 0.0)

    return pl.pallas_call(
        body,
        grid=(_N // _R,),
        in_specs=[
            pl.BlockSpec((_NC, _R, 128), lambda g: (0, g, 0)),
            pl.BlockSpec((_R, 128), lambda g: (g, 0)),
            pl.BlockSpec((_NC, _R, 16), lambda g: (0, g, 0)),
            pl.BlockSpec((1, 128), lambda g: (0, 0)),
        ],
        out_specs=pl.BlockSpec((_R, 128), lambda g: (g, 0)),
        out_shape=jax.ShapeDtypeStruct((_N, 128), jnp.float32),
    )(agg2, hs2, degp, b2r)


def kernel(x, edge_index, W1, b1, W2, b2):
    src = edge_index[0]
    dst = edge_index[1]
    ept_deg = _E // (_NC * _NS)
    dst_deg = dst.reshape(_NC * _NS, ept_deg // _K, _K)
    ept1 = _E // _NS
    src_l1 = jnp.stack([src, src + _N]).reshape(_NC, _NS, ept1 // _K, _K)
    dst_l1 = jnp.stack([dst, dst]).reshape(_NC, _NS, ept1 // _K, _K)
    ept2 = _E // (_NC * _NS)
    src_l2 = src.reshape(_NC, _NS, ept2 // _K, _K)
    dst_l2 = dst.reshape(_NC, _NS, ept2 // _K, _K)

    lane = jnp.arange(16)
    ones_rows = jnp.where(lane[None, :] == 0, 1.0, 0.0).astype(jnp.float32)
    ones_rows = jnp.broadcast_to(ones_rows, (_K, 16))
    zeros_nd = jnp.zeros((_N, 128), jnp.float32)

    degp = _deg_k(dst_deg, ones_rows, zeros_nd)
    hs1 = _tc1(x, W1, degp)
    agg1 = _spmm_l1(hs1.reshape(_NC * _N, 128), src_l1, dst_l1, zeros_nd)
    hs2 = _tc2(agg1, hs1, degp, b1.reshape(2, 128), W2.reshape(2, 128, 128))
    agg2 = _spmm_l2(hs2, src_l2, dst_l2, zeros_nd)
    return _tc3(agg2, hs2, degp, b2.reshape(1, 128))


# speedup vs baseline: 12.1362x; 12.1362x over previous
import functools

import jax
import jax.numpy as jnp
from jax import lax
from jax.experimental import pallas as pl
from jax.experimental.pallas import tpu as pltpu
from jax.experimental.pallas import tpu_sc as plsc

_N = 10000
_E = 320000
_NC = 2
_NS = 16
_K = 80
_R = 1000
_NP = 10240
_RPT = _NP // _NS


def _make_spmm(table_rows, epc):
    ept = epc // _NS
    nch = ept // _K

    @functools.partial(
        pl.kernel,
        out_type=jax.ShapeDtypeStruct((_NC, _NP, 128), jnp.float32),
        mesh=plsc.VectorSubcoreMesh(core_axis_name="c", subcore_axis_name="s",
                                    num_cores=_NC, num_subcores=_NS),
        scratch_types=[
            pltpu.VMEM((nch, _K), jnp.int32),
            pltpu.VMEM((nch, _K), jnp.int32),
            pltpu.VMEM((_K, 128), jnp.float32),
            pltpu.VMEM_SHARED((_NP, 128), jnp.float32),
            pltpu.SemaphoreType.DMA,
        ],
    )
    def spmm(tab_hbm, src_hbm, dst_hbm, zeros_hbm, out_hbm,
             idx_s, idx_d, rows, acc, sem):
        c = lax.axis_index("c")
        s = lax.axis_index("s")
        pltpu.sync_copy(zeros_hbm.at[pl.ds(s * _RPT, _RPT)],
                        acc.at[pl.ds(s * _RPT, _RPT)])
        pltpu.sync_copy(src_hbm.at[c, s], idx_s)
        pltpu.sync_copy(dst_hbm.at[c, s], idx_d)
        plsc.subcore_barrier()

        def chunk(i, carry):
            pltpu.async_copy(tab_hbm.at[idx_s.at[i]], rows, sem).wait()
            pltpu.sync_copy(rows, acc.at[idx_d.at[i]], add=True)
            return carry

        lax.fori_loop(0, nch, chunk, 0)
        plsc.subcore_barrier()
        pltpu.sync_copy(acc.at[pl.ds(s * _RPT, _RPT)],
                        out_hbm.at[c, pl.ds(s * _RPT, _RPT)])

    return spmm


@functools.lru_cache(maxsize=None)
def _sc_kernels():
    return _make_spmm(_N, _E // _NC)


def _dinv_of(degp_blk):
    deg = degp_blk[0, :, 0] + degp_blk[1, :, 0] + 1.0
    return lax.rsqrt(deg)[:, None]


def _tc1(x, W1, degp):
    def body(x_ref, w_ref, degp_ref, out_ref):
        dinv = _dinv_of(degp_ref[...])
        h = jnp.dot(x_ref[...], w_ref[...], preferred_element_type=jnp.float32)
        hs = h * dinv
        out_ref[0] = hs[:, :128]
        out_ref[1] = hs[:, 128:]

    return pl.pallas_call(
        body,
        grid=(_N // _R,),
        in_specs=[
            pl.BlockSpec((_R, 128), lambda g: (g, 0)),
            pl.BlockSpec((128, 256), lambda g: (0, 0)),
            pl.BlockSpec((_NC, _R, 128), lambda g: (0, g, 0)),
        ],
        out_specs=pl.BlockSpec((_NC, _R, 128), lambda g: (0, g, 0)),
        out_shape=jax.ShapeDtypeStruct((_NC, _N, 128), jnp.float32),
    )(x, W1, degp)


def _tc2(agg1_0, agg1_1, hs1, degp, b1r, W2r):
    def body(a0_ref, a1_ref, hs_ref, degp_ref, b_ref, w_ref, out_ref):
        dinv = _dinv_of(degp_ref[...])
        y0 = jnp.maximum(
            dinv * (a0_ref[0] + a0_ref[1] + hs_ref[0]) + b_ref[0], 0.0)
        y1 = jnp.maximum(
            dinv * (a1_ref[0] + a1_ref[1] + hs_ref[1]) + b_ref[1], 0.0)
        h2 = (jnp.dot(y0, w_ref[0], preferred_element_type=jnp.float32)
              + jnp.dot(y1, w_ref[1], preferred_element_type=jnp.float32))
        out_ref[...] = h2 * dinv

    return pl.pallas_call(
        body,
        grid=(_N // _R,),
        in_specs=[
            pl.BlockSpec((_NC, _R, 128), lambda g: (0, g, 0)),
            pl.BlockSpec((_NC, _R, 128), lambda g: (0, g, 0)),
            pl.BlockSpec((_NC, _R, 128), lambda g: (0, g, 0)),
            pl.BlockSpec((_NC, _R, 128), lambda g: (0, g, 0)),
            pl.BlockSpec((2, 128), lambda g: (0, 0)),
            pl.BlockSpec((2, 128, 128), lambda g: (0, 0, 0)),
        ],
        out_specs=pl.BlockSpec((_R, 128), lambda g: (g, 0)),
        out_shape=jax.ShapeDtypeStruct((_N, 128), jnp.float32),
    )(agg1_0, agg1_1, hs1, degp, b1r, W2r)


def _tc3(agg2, hs2, degp, b2r):
    def body(agg_ref, hs_ref, degp_ref, b_ref, out_ref):
        dinv = _dinv_of(degp_ref[...])
        msg = agg_ref[0] + agg_ref[1] + hs_ref[...]
        out_ref[...] = jnp.maximum(dinv * msg + b_ref[0], 0.0)

    return pl.pallas_call(
        body,
        grid=(_N // _R,),
        in_specs=[
            pl.BlockSpec((_NC, _R, 128), lambda g: (0, g, 0)),
            pl.BlockSpec((_R, 128), lambda g: (g, 0)),
            pl.BlockSpec((_NC, _R, 128), lambda g: (0, g, 0)),
            pl.BlockSpec((1, 128), lambda g: (0, 0)),
        ],
        out_specs=pl.BlockSpec((_R, 128), lambda g: (g, 0)),
        out_shape=jax.ShapeDtypeStruct((_N, 128), jnp.float32),
    )(agg2, hs2, degp, b2r)


def kernel(x, edge_index, W1, b1, W2, b2):
    src = edge_index[0]
    dst = edge_index[1]
    ept = _E // (_NC * _NS)
    src_es = src.reshape(_NC, _NS, ept // _K, _K)
    dst_es = dst.reshape(_NC, _NS, ept // _K, _K)

    ones_nd = jnp.ones((_N, 128), jnp.float32)
    zeros_nd = jnp.zeros((_NP, 128), jnp.float32)

    _spmm = _sc_kernels()
    degp = _spmm(ones_nd, src_es, dst_es, zeros_nd)
    hs1 = _tc1(x, W1, degp)
    agg1_0 = _spmm(hs1[0], src_es, dst_es, zeros_nd)
    agg1_1 = _spmm(hs1[1], src_es, dst_es, zeros_nd)
    hs2 = _tc2(agg1_0, agg1_1, hs1, degp,
               b1.reshape(2, 128), W2.reshape(2, 128, 128))
    agg2 = _spmm(hs2, src_es, dst_es, zeros_nd)
    return _tc3(agg2, hs2, degp, b2.reshape(1, 128))
